# baseline (speedup 1.0000x reference)
import functools

import jax
import jax.numpy as jnp
from jax import lax
from jax.experimental import pallas as pl
from jax.experimental.pallas import tpu as pltpu
from jax.experimental.pallas import tpu_sc as plsc

_NC = 2
_NS = 16
_NW = _NC * _NS
_LANES = 16
_TILE_R = 8
_CHUNK = 64


@functools.partial(jax.jit, static_argnames=("n_rows", "d", "seq_len"))
def _sc_embed(table3, idx2d, pos, *, n_rows, d, seq_len):
    b_per_w = n_rows // _NW
    rows_per_w = b_per_w // 128
    n_chunks = b_per_w // _CHUNK
    n_sg = _CHUNK // _LANES
    h = d // 2

    mesh = plsc.VectorSubcoreMesh(
        core_axis_name="c", subcore_axis_name="s",
        num_cores=_NC, num_subcores=_NS)

    scratch = [
        pltpu.VMEM((rows_per_w, 128), jnp.int32),
        pltpu.VMEM((seq_len, d), jnp.float32),
        pltpu.VMEM((_CHUNK,), jnp.int32),
        pltpu.VMEM((_CHUNK,), jnp.int32),
        pltpu.VMEM((_CHUNK, _TILE_R, d), jnp.float32),
        pltpu.VMEM((_CHUNK, _TILE_R, d), jnp.float32),
        pltpu.VMEM((_CHUNK, h), jnp.float32),
        pltpu.VMEM((_CHUNK, h), jnp.float32),
        pltpu.VMEM((_CHUNK, h), jnp.float32),
        pltpu.VMEM((_CHUNK, h), jnp.float32),
        pltpu.SemaphoreType.DMA,
        pltpu.SemaphoreType.DMA,
        pltpu.SemaphoreType.DMA,
        pltpu.SemaphoreType.DMA,
    ]

    @functools.partial(
        pl.kernel,
        out_type=(jax.ShapeDtypeStruct((n_rows, h), jnp.float32),
                  jax.ShapeDtypeStruct((n_rows, h), jnp.float32)),
        mesh=mesh,
        scratch_types=scratch,
        compiler_params=pltpu.CompilerParams(
            use_tc_tiling_on_sc=False, needs_layout_passes=False),
    )
    def k(table_hbm, idx_hbm, pos_hbm, re_hbm, im_hbm,
          idx_v, pos_v, tid0, tid1, bank0, bank1,
          sre0, sim0, sre1, sim1, semb0, semb1, semw0, semw1):
        tid = (tid0, tid1)
        bank = (bank0, bank1)
        stg = ((sre0, sim0), (sre1, sim1))
        semb = (semb0, semb1)
        semw = (semw0, semw1)
        wid = lax.axis_index("s") * _NC + lax.axis_index("c")
        base = wid * b_per_w
        pltpu.sync_copy(idx_hbm.at[pl.ds(wid * rows_per_w, rows_per_w), :],
                        idx_v)
        pltpu.sync_copy(pos_hbm, pos_v)

        iota = lax.iota(jnp.int32, _LANES)

        def chunk_vec(cc, sg):
            return idx_v[cc >> 1, pl.ds((cc & 1) * _CHUNK + sg * _LANES,
                                        _LANES)]

        def fire(cc, b):
            for sg in range(n_sg):
                tid[b][pl.ds(sg * _LANES, _LANES)] = (
                    jnp.right_shift(chunk_vec(cc, sg), 3))
            pltpu.async_copy(table_hbm.at[tid[b]], bank[b], semb[b])

        def drain_bank(b):
            pltpu.make_async_copy(table_hbm.at[pl.ds(0, _CHUNK), :, :],
                                  bank[b], semb[b]).wait()

        def process(cc, b):
            for sg in range(n_sg):
                vec = chunk_vec(cc, sg)
                rvec = jnp.bitwise_and(vec, 7)
                slots = jnp.full((_LANES,), sg * _LANES, jnp.int32) + iota
                lbase = jnp.bitwise_and(cc * _CHUNK + sg * _LANES,
                                        seq_len - 1)
                lvec = jnp.full((_LANES,), 0, jnp.int32) + lbase + iota
                for c in range(d):
                    cvec = jnp.full((_LANES,), c, jnp.int32)
                    tv = plsc.load_gather(bank[b], [slots, rvec, cvec])
                    pv = plsc.load_gather(pos_v, [lvec, cvec])
                    sv = tv + pv
                    if c < h:
                        plsc.store_scatter(
                            stg[b][0],
                            [slots, cvec], sv)
                    else:
                        plsc.store_scatter(
                            stg[b][1],
                            [slots, cvec - h], sv)

        def stage_out(cc, b):
            dst = pl.ds(base + cc * _CHUNK, _CHUNK)
            pltpu.async_copy(stg[b][0], re_hbm.at[dst, :], semw[b])
            pltpu.async_copy(stg[b][1], im_hbm.at[dst, :], semw[b])

        def drain_stage(b):
            for sref in (stg[b][0], stg[b][1]):
                pltpu.make_async_copy(
                    sref, re_hbm.at[pl.ds(0, _CHUNK), :], semw[b]).wait()

        fire(0, 0)

        def body(m, _):
            c0 = m * 2
            fire(c0 + 1, 1)
            drain_bank(0)

            @pl.when(m >= 1)
            def _():
                drain_stage(0)

            process(c0, 0)
            stage_out(c0, 0)

            @pl.when(m < n_chunks // 2 - 1)
            def _():
                fire(c0 + 2, 0)

            drain_bank(1)

            @pl.when(m >= 1)
            def _():
                drain_stage(1)

            process(c0 + 1, 1)
            stage_out(c0 + 1, 1)
            return 0

        lax.fori_loop(0, n_chunks // 2, body, 0)
        drain_stage(0)
        drain_stage(1)

    return k(table3, idx2d, pos)


def kernel(x, token_table, pos_embedding):
    B, L = x.shape
    d = token_table.shape[1]
    n_rows = B * L
    idx2d = x.reshape(n_rows // 128, 128).astype(jnp.int32)
    pos = pos_embedding[0, :L, :]
    table3 = token_table.reshape(-1, _TILE_R, d)
    re, im = _sc_embed(table3, idx2d, pos, n_rows=n_rows, d=d, seq_len=L)
    re = re.reshape(B, L, d // 2)
    im = im.reshape(B, L, d // 2)
    return jax.lax.complex(re, im)


# speedup vs baseline: 1.0074x; 1.0074x over previous
import functools

import jax
import jax.numpy as jnp
from jax import lax
from jax.experimental import pallas as pl
from jax.experimental.pallas import tpu as pltpu
from jax.experimental.pallas import tpu_sc as plsc

_NC = 2
_NS = 16
_NW = _NC * _NS
_LANES = 16
_TILE_R = 8


@functools.partial(jax.jit, static_argnames=("n_rows", "d", "seq_len"))
def _sc_embed(table3, idx2d, pos, *, n_rows, d, seq_len):
    b_per_w = n_rows // _NW
    rows_per_w = b_per_w // 128
    n_groups = b_per_w // _LANES
    h = d // 2

    mesh = plsc.VectorSubcoreMesh(
        core_axis_name="c", subcore_axis_name="s",
        num_cores=_NC, num_subcores=_NS)

    scratch = [
        pltpu.VMEM((rows_per_w, 128), jnp.int32),
        pltpu.VMEM((seq_len, d), jnp.float32),
    ]
    scratch += [pltpu.VMEM((_LANES,), jnp.int32)] * (2 * _LANES)
    scratch += [
        pltpu.VMEM((_LANES, _TILE_R, d), jnp.float32),
        pltpu.VMEM((_LANES, _TILE_R, d), jnp.float32),
        pltpu.VMEM((_LANES, h), jnp.float32),
        pltpu.VMEM((_LANES, h), jnp.float32),
        pltpu.VMEM((_LANES, h), jnp.float32),
        pltpu.VMEM((_LANES, h), jnp.float32),
        pltpu.SemaphoreType.DMA,
        pltpu.SemaphoreType.DMA,
        pltpu.SemaphoreType.DMA,
        pltpu.SemaphoreType.DMA,
    ]

    @functools.partial(
        pl.kernel,
        out_type=(jax.ShapeDtypeStruct((n_rows, h), jnp.float32),
                  jax.ShapeDtypeStruct((n_rows, h), jnp.float32)),
        mesh=mesh,
        scratch_types=scratch,
        compiler_params=pltpu.CompilerParams(
            use_tc_tiling_on_sc=False, needs_layout_passes=False),
    )
    def k(table_hbm, idx_hbm, pos_hbm, re_hbm, im_hbm,
          idx_v, pos_v, *rest):
        tid = (rest[:_LANES], rest[_LANES:2 * _LANES])
        (bank0, bank1, sre0, sim0, sre1, sim1,
         semb0, semb1, semw0, semw1) = rest[2 * _LANES:]
        bank = (bank0, bank1)
        stg = ((sre0, sim0), (sre1, sim1))
        semb = (semb0, semb1)
        semw = (semw0, semw1)
        wid = lax.axis_index("s") * _NC + lax.axis_index("c")
        base = wid * b_per_w
        pltpu.sync_copy(idx_hbm.at[pl.ds(wid * rows_per_w, rows_per_w), :],
                        idx_v)
        pltpu.sync_copy(pos_hbm, pos_v)

        iota = lax.iota(jnp.int32, _LANES)

        def group_vec(g):
            return idx_v[g >> 3, pl.ds((g & 7) * _LANES, _LANES)]

        def fire(g, b):
            tids = jnp.right_shift(group_vec(g), 3)
            for lane in range(_LANES):
                perm = jnp.bitwise_and(iota + lane, _LANES - 1)
                tid[b][lane][...] = tids[perm]
                pltpu.async_copy(table_hbm.at[tid[b][lane].at[pl.ds(0, 1)]],
                                 bank[b].at[pl.ds(lane, 1), :, :], semb[b])

        def drain_bank(b):
            pltpu.make_async_copy(table_hbm.at[pl.ds(0, _LANES), :, :],
                                  bank[b], semb[b]).wait()

        def process(g, b):
            vec = group_vec(g)
            rvec = jnp.bitwise_and(vec, 7)
            lbase = jnp.bitwise_and(g * _LANES, seq_len - 1)
            lvec = jnp.full((_LANES,), 0, jnp.int32) + lbase + iota
            for c in range(d):
                cvec = jnp.full((_LANES,), c, jnp.int32)
                tv = plsc.load_gather(bank[b], [iota, rvec, cvec])
                pv = plsc.load_gather(pos_v, [lvec, cvec])
                sv = tv + pv
                if c < h:
                    plsc.store_scatter(stg[b][0], [iota, cvec], sv)
                else:
                    plsc.store_scatter(stg[b][1], [iota, cvec - h], sv)

        def stage_out(g, b):
            dst = pl.ds(base + g * _LANES, _LANES)
            pltpu.async_copy(stg[b][0], re_hbm.at[dst, :], semw[b])
            pltpu.async_copy(stg[b][1], im_hbm.at[dst, :], semw[b])

        def drain_stage(b):
            for sref in (stg[b][0], stg[b][1]):
                pltpu.make_async_copy(
                    sref, re_hbm.at[pl.ds(0, _LANES), :], semw[b]).wait()

        fire(0, 0)

        def body(m, _):
            g0 = m * 2
            fire(g0 + 1, 1)
            drain_bank(0)

            @pl.when(m >= 1)
            def _():
                drain_stage(0)

            process(g0, 0)
            stage_out(g0, 0)

            @pl.when(m < n_groups // 2 - 1)
            def _():
                fire(g0 + 2, 0)

            drain_bank(1)

            @pl.when(m >= 1)
            def _():
                drain_stage(1)

            process(g0 + 1, 1)
            stage_out(g0 + 1, 1)
            return 0

        lax.fori_loop(0, n_groups // 2, body, 0)
        drain_stage(0)
        drain_stage(1)

    return k(table3, idx2d, pos)


def kernel(x, token_table, pos_embedding):
    B, L = x.shape
    d = token_table.shape[1]
    n_rows = B * L
    idx2d = x.reshape(n_rows // 128, 128).astype(jnp.int32)
    pos = pos_embedding[0, :L, :]
    table3 = token_table.reshape(-1, _TILE_R, d)
    re, im = _sc_embed(table3, idx2d, pos, n_rows=n_rows, d=d, seq_len=L)
    re = re.reshape(B, L, d // 2)
    im = im.reshape(B, L, d // 2)
    return jax.lax.complex(re, im)
